# baseline (speedup 1.0000x reference)
import functools

import jax
import jax.numpy as jnp
from jax import lax
from jax.experimental import pallas as pl
from jax.experimental.pallas import tpu as pltpu
from jax.experimental.pallas import tpu_sc as plsc

NC = 2
NS = 16
L = 16
NW = NC * NS

EQ = 4
TERMS = 64
ROWS = 65536
NPAIR = EQ * TERMS
PAIR_PER_W = NPAIR // NW
NSEM = 4

_mesh = plsc.VectorSubcoreMesh(core_axis_name="c", subcore_axis_name="s")


@functools.partial(
    pl.kernel,
    out_type=jax.ShapeDtypeStruct((NPAIR, ROWS), jnp.float32),
    mesh=_mesh,
    scratch_types=[
        pltpu.VMEM((EQ, TERMS), jnp.int32),
        pltpu.SemaphoreType.DMA,
        pltpu.SemaphoreType.DMA,
        pltpu.SemaphoreType.DMA,
        pltpu.SemaphoreType.DMA,
    ],
    compiler_params=pltpu.CompilerParams(needs_layout_passes=False),
)
def _row_gather(theta_hbm, masks_hbm, out_hbm, masks_v, s0, s1, s2, s3):
    wid = lax.axis_index("s") * NC + lax.axis_index("c")
    sems = (s0, s1, s2, s3)

    pltpu.sync_copy(masks_hbm, masks_v)

    row = wid // (NW // EQ)
    cblk = (wid % (NW // EQ)) // 2
    o = (wid % 2) * PAIR_PER_W
    lanes = lax.iota(jnp.int32, L)
    win = jnp.zeros((L,), jnp.int32)
    for i in range(EQ):
        for c in range(TERMS // L):
            win = jnp.where(
                jnp.logical_and(row == i, cblk == c),
                masks_v[i, pl.ds(c * L, L)], win)

    def copy_row(j):
        m = jnp.sum(jnp.where(lanes == o + j, win, 0))
        drow = wid * PAIR_PER_W + j
        return pltpu.make_async_copy(
            theta_hbm.at[pl.ds(m, 1)],
            out_hbm.at[pl.ds(drow, 1)], sems[j % NSEM])

    copies = [copy_row(j) for j in range(PAIR_PER_W)]
    for c in copies:
        c.start()
    for c in copies:
        c.wait()


def kernel(theta, coeffs, masks):
    out_flat = _row_gather(theta.T, masks)
    sparse_theta = jnp.transpose(
        out_flat.reshape(EQ, TERMS, ROWS), (0, 2, 1))
    return (sparse_theta, coeffs)


# speedup vs baseline: 38.6760x; 38.6760x over previous
import functools

import jax
import jax.numpy as jnp
from jax import lax
from jax.experimental import pallas as pl
from jax.experimental.pallas import tpu as pltpu
from jax.experimental.pallas import tpu_sc as plsc

NC = 2
NS = 16
L = 16
NW = NC * NS

EQ = 4
TERMS = 64
ROWS = 65536
NPAIR = EQ * TERMS
T_PER_W = TERMS // NW
QCHUNK = 4
CW = ROWS // QCHUNK
NBA = 4
GROUPS_W = T_PER_W * QCHUNK

_mesh = plsc.VectorSubcoreMesh(core_axis_name="c", subcore_axis_name="s")


@functools.partial(
    pl.kernel,
    out_type=jax.ShapeDtypeStruct((NPAIR, ROWS), jnp.float32),
    mesh=_mesh,
    scratch_types=[
        pltpu.VMEM((EQ, TERMS), jnp.int32),
        pltpu.VMEM((EQ, L), jnp.int32),
        pltpu.VMEM((NBA, 1, CW), jnp.float32),
        pltpu.VMEM((EQ - 1, 1, CW), jnp.float32),
        pltpu.SemaphoreType.DMA,
        pltpu.SemaphoreType.DMA,
        pltpu.SemaphoreType.DMA,
        pltpu.SemaphoreType.DMA,
        pltpu.SemaphoreType.DMA,
        pltpu.SemaphoreType.DMA,
        pltpu.SemaphoreType.DMA,
        pltpu.SemaphoreType.DMA,
        pltpu.SemaphoreType.DMA,
        pltpu.SemaphoreType.DMA,
        pltpu.SemaphoreType.DMA,
        pltpu.SemaphoreType.DMA,
        pltpu.SemaphoreType.DMA,
        pltpu.SemaphoreType.DMA,
    ],
    compiler_params=pltpu.CompilerParams(needs_layout_passes=False),
)
def _row_gather(theta_hbm, masks_hbm, out_hbm,
                win_v, idx_v, bufa_v, bufu_v,
                ga0, ga1, ga2, ga3, wa0, wa1, wa2, wa3,
                gu1, gu2, gu3, wu1, wu2, wu3):
    wid = lax.axis_index("s") * NC + lax.axis_index("c")
    ga = (ga0, ga1, ga2, ga3)
    wa = (wa0, wa1, wa2, wa3)
    gu = (None, gu1, gu2, gu3)
    wu = (None, wu1, wu2, wu3)

    wpw = NW // (TERMS // L)
    wsel = wid // wpw
    o = (wid % wpw) * T_PER_W
    pltpu.sync_copy(masks_hbm, win_v)

    lanes = lax.iota(jnp.int32, L)
    sel = jnp.minimum(o + lanes, L - 1)
    dnums = lax.GatherDimensionNumbers(
        offset_dims=(), collapsed_slice_dims=(0,), start_index_map=(0,))
    for i in range(EQ):
        win = jnp.zeros((L,), jnp.int32)
        for c in range(TERMS // L):
            win = jnp.where(wsel == c, win_v[i, pl.ds(c * L, L)], win)
        idx_v[i, :] = lax.gather(
            win, sel[:, None], dnums, (1,),
            mode=lax.GatherScatterMode.PROMISE_IN_BOUNDS)

    v0 = idx_v[0, :]
    conds = [None]
    for i in range(1, EQ):
        d = jnp.abs(idx_v[i, :] - v0)
        s = jnp.sum(jnp.where(lanes < T_PER_W, d, 0))
        conds.append(s == 0)

    def gather_a(g):
        t, q, b = g // QCHUNK, g % QCHUNK, g % NBA
        return pltpu.make_async_copy(
            theta_hbm.at[idx_v.at[0, pl.ds(t, 1)], pl.ds(q * CW, CW)],
            bufa_v.at[b], ga[b])

    def gather_u(i, g):
        t, q = g // QCHUNK, g % QCHUNK
        return pltpu.make_async_copy(
            theta_hbm.at[idx_v.at[i, pl.ds(t, 1)], pl.ds(q * CW, CW)],
            bufu_v.at[i - 1], gu[i])

    def write_out(i, g, src_ref, sem):
        t, q = g // QCHUNK, g % QCHUNK
        drow = i * TERMS + wid * T_PER_W + t
        return pltpu.make_async_copy(
            src_ref, out_hbm.at[pl.ds(drow, 1), pl.ds(q * CW, CW)], sem)

    def drain_group(g):
        b = g % NBA
        write_out(0, g, bufa_v.at[b], wa[b]).wait()
        for i in range(1, EQ):
            @pl.when(conds[i])
            def _(i=i, g=g, b=b):
                write_out(i, g, bufa_v.at[b], wa[b]).wait()

    for s in range(NBA - 2):
        gather_a(s).start()
    for g in range(GROUPS_W):
        b = g % NBA
        gather_a(g).wait()
        write_out(0, g, bufa_v.at[b], wa[b]).start()
        for i in range(1, EQ):
            @pl.when(conds[i])
            def _(i=i, g=g, b=b):
                write_out(i, g, bufa_v.at[b], wa[b]).start()

            @pl.when(jnp.logical_not(conds[i]))
            def _(i=i, g=g):
                if g > 0:
                    write_out(i, g - 1, bufu_v.at[i - 1], wu[i]).wait()
                gather_u(i, g).start()
                gather_u(i, g).wait()
                write_out(i, g, bufu_v.at[i - 1], wu[i]).start()

        s = g + NBA - 2
        if s < GROUPS_W:
            if s - NBA >= 0:
                drain_group(s - NBA)
            gather_a(s).start()

    for g in range(GROUPS_W - NBA, GROUPS_W):
        drain_group(g)
    for i in range(1, EQ):
        @pl.when(jnp.logical_not(conds[i]))
        def _(i=i):
            write_out(i, GROUPS_W - 1, bufu_v.at[i - 1], wu[i]).wait()


def kernel(theta, coeffs, masks):
    out_flat = _row_gather(theta.T, masks)
    sparse_theta = jnp.transpose(
        out_flat.reshape(EQ, TERMS, ROWS), (0, 2, 1))
    return (sparse_theta, coeffs)


# speedup vs baseline: 38.7474x; 1.0018x over previous
import functools

import jax
import jax.numpy as jnp
from jax import lax
from jax.experimental import pallas as pl
from jax.experimental.pallas import tpu as pltpu
from jax.experimental.pallas import tpu_sc as plsc

NC = 2
NS = 16
L = 16
NW = NC * NS

EQ = 4
TERMS = 64
ROWS = 65536
NPAIR = EQ * TERMS
T_PER_W = TERMS // NW
QCHUNK = 4
CW = ROWS // QCHUNK
NBA = 5
GROUPS_W = T_PER_W * QCHUNK

_mesh = plsc.VectorSubcoreMesh(core_axis_name="c", subcore_axis_name="s")


@functools.partial(
    pl.kernel,
    out_type=jax.ShapeDtypeStruct((NPAIR, ROWS), jnp.float32),
    mesh=_mesh,
    scratch_types=[
        pltpu.VMEM((EQ, TERMS), jnp.int32),
        pltpu.VMEM((EQ, L), jnp.int32),
        pltpu.VMEM((NBA, 1, CW), jnp.float32),
        pltpu.VMEM((1, 1, CW), jnp.float32),
        pltpu.SemaphoreType.DMA,
        pltpu.SemaphoreType.DMA,
        pltpu.SemaphoreType.DMA,
        pltpu.SemaphoreType.DMA,
        pltpu.SemaphoreType.DMA,
        pltpu.SemaphoreType.DMA,
        pltpu.SemaphoreType.DMA,
        pltpu.SemaphoreType.DMA,
        pltpu.SemaphoreType.DMA,
        pltpu.SemaphoreType.DMA,
        pltpu.SemaphoreType.DMA,
        pltpu.SemaphoreType.DMA,
    ],
    compiler_params=pltpu.CompilerParams(needs_layout_passes=False),
)
def _row_gather(theta_hbm, masks_hbm, out_hbm,
                win_v, idx_v, bufa_v, bufu_v,
                ga0, ga1, ga2, ga3, ga4, wa0, wa1, wa2, wa3, wa4,
                gu_s, wu_s):
    wid = lax.axis_index("s") * NC + lax.axis_index("c")
    ga = (ga0, ga1, ga2, ga3, ga4)
    wa = (wa0, wa1, wa2, wa3, wa4)

    wpw = NW // (TERMS // L)
    wsel = wid // wpw
    o = (wid % wpw) * T_PER_W
    pltpu.sync_copy(masks_hbm, win_v)

    lanes = lax.iota(jnp.int32, L)
    sel = jnp.minimum(o + lanes, L - 1)
    dnums = lax.GatherDimensionNumbers(
        offset_dims=(), collapsed_slice_dims=(0,), start_index_map=(0,))
    for i in range(EQ):
        win = jnp.zeros((L,), jnp.int32)
        for c in range(TERMS // L):
            win = jnp.where(wsel == c, win_v[i, pl.ds(c * L, L)], win)
        idx_v[i, :] = lax.gather(
            win, sel[:, None], dnums, (1,),
            mode=lax.GatherScatterMode.PROMISE_IN_BOUNDS)

    v0 = idx_v[0, :]
    conds = [None]
    for i in range(1, EQ):
        d = jnp.abs(idx_v[i, :] - v0)
        s = jnp.sum(jnp.where(lanes < T_PER_W, d, 0))
        conds.append(s == 0)

    def gather_a(g):
        t, q, b = g // QCHUNK, g % QCHUNK, g % NBA
        return pltpu.make_async_copy(
            theta_hbm.at[idx_v.at[0, pl.ds(t, 1)], pl.ds(q * CW, CW)],
            bufa_v.at[b], ga[b])

    def gather_u(i, g):
        t, q = g // QCHUNK, g % QCHUNK
        return pltpu.make_async_copy(
            theta_hbm.at[idx_v.at[i, pl.ds(t, 1)], pl.ds(q * CW, CW)],
            bufu_v.at[0], gu_s)

    def write_out(i, g, src_ref, sem):
        t, q = g // QCHUNK, g % QCHUNK
        drow = i * TERMS + wid * T_PER_W + t
        return pltpu.make_async_copy(
            src_ref, out_hbm.at[pl.ds(drow, 1), pl.ds(q * CW, CW)], sem)

    def drain_group(g):
        b = g % NBA
        write_out(0, g, bufa_v.at[b], wa[b]).wait()
        for i in range(1, EQ):
            @pl.when(conds[i])
            def _(i=i, g=g, b=b):
                write_out(i, g, bufa_v.at[b], wa[b]).wait()

    for s in range(NBA - 2):
        gather_a(s).start()
    for g in range(GROUPS_W):
        b = g % NBA
        gather_a(g).wait()
        write_out(0, g, bufa_v.at[b], wa[b]).start()
        for i in range(1, EQ):
            @pl.when(conds[i])
            def _(i=i, g=g, b=b):
                write_out(i, g, bufa_v.at[b], wa[b]).start()

            @pl.when(jnp.logical_not(conds[i]))
            def _(i=i, g=g):
                gather_u(i, g).start()
                gather_u(i, g).wait()
                write_out(i, g, bufu_v.at[0], wu_s).start()
                write_out(i, g, bufu_v.at[0], wu_s).wait()

        s = g + NBA - 2
        if s < GROUPS_W:
            if s - NBA >= 0:
                drain_group(s - NBA)
            gather_a(s).start()

    for g in range(GROUPS_W - NBA, GROUPS_W):
        drain_group(g)


def kernel(theta, coeffs, masks):
    out_flat = _row_gather(theta.T, masks)
    sparse_theta = jnp.transpose(
        out_flat.reshape(EQ, TERMS, ROWS), (0, 2, 1))
    return (sparse_theta, coeffs)
